# baseline (speedup 1.0000x reference)
import functools

import jax
import jax.numpy as jnp
from jax import lax
from jax.experimental import pallas as pl
from jax.experimental.pallas import tpu as pltpu
from jax.experimental.pallas import tpu_sc as plsc

NC = 2
NS = 16
NW = NC * NS

CHUNK = 128
NBUF = 4


def _gather_body(n_chunks, ids_hbm, table_hbm, out_hbm, idx_v, rows_v,
                 in_sems, out_sems):
  wid = lax.axis_index("s") * NC + lax.axis_index("c")
  chunk0 = wid * n_chunks
  row0 = chunk0 * CHUNK

  pltpu.sync_copy(ids_hbm.at[pl.ds(chunk0, n_chunks)], idx_v)

  def start_gather(g, b):
    pltpu.async_copy(table_hbm.at[idx_v.at[g]], rows_v.at[b],
                     in_sems.at[b])

  for b in range(NBUF):
    start_gather(b, b)

  def round_body(r, _):
    for b in range(NBUF):
      g = r * NBUF + b
      pltpu.make_async_copy(table_hbm.at[idx_v.at[g]], rows_v.at[b],
                            in_sems.at[b]).wait()
      out = pltpu.async_copy(
          rows_v.at[b], out_hbm.at[pl.ds(row0 + g * CHUNK, CHUNK)],
          out_sems.at[b])
      out.wait()

      @pl.when(g + NBUF < n_chunks)
      def _():
        start_gather(g + NBUF, b)

    return _

  lax.fori_loop(0, n_chunks // NBUF, round_body, None)


@jax.jit
def _token_gather(ids2d, w_emb):
  n_flat = ids2d.shape[0] * ids2d.shape[1]
  n_chunks = n_flat // (NW * CHUNK)
  hidden = w_emb.shape[1]
  mesh = plsc.VectorSubcoreMesh(core_axis_name="c", subcore_axis_name="s",
                                num_cores=NC, num_subcores=NS)
  fn = pl.kernel(
      functools.partial(_gather_body, n_chunks),
      out_type=jax.ShapeDtypeStruct((n_flat, hidden), jnp.float32),
      mesh=mesh,
      scratch_types=[
          pltpu.VMEM((n_chunks, CHUNK), jnp.int32),
          pltpu.VMEM((NBUF, CHUNK, hidden), jnp.float32),
          pltpu.SemaphoreType.DMA((NBUF,)),
          pltpu.SemaphoreType.DMA((NBUF,)),
      ],
      compiler_params=pltpu.CompilerParams(use_tc_tiling_on_sc=False),
  )
  return fn(ids2d, w_emb)


def kernel(input_ids, latents, W_emb):
  del latents
  b, l = input_ids.shape
  ids2d = input_ids.reshape(-1, CHUNK).astype(jnp.int32)
  out = _token_gather(ids2d, W_emb)
  return out.reshape(b, l, W_emb.shape[1])
